# Initial kernel scaffold; baseline (speedup 1.0000x reference)
import jax
import jax.numpy as jnp
from jax.experimental import pallas as pl


def kernel(x, edge_index, Wl0, bl0, Wr0, gamma0, beta0, Wl1, bl1, Wr1, gamma1, beta1, Wl2, bl2, Wr2, gamma2, beta2, Wlin, blin):
    raise NotImplementedError("write your pallas kernel here")


# speedup vs baseline: 4.0662x; 4.0662x over previous
import functools

import jax
import jax.numpy as jnp
from jax import lax
from jax.experimental import pallas as pl
from jax.experimental.pallas import tpu as pltpu
from jax.experimental.pallas import tpu_sc as plsc

N = 10000
E = 320000
HID = 128
HH = HID // 2
EPS = 1e-5

NC = 2
NS = 16
W = 72

BATCH = 128
NBUF = 4
NB = 160
E_PAD = NS * NB * BATCH
N_AGG = 10112
ZROWS = N_AGG // NS

_sc_mesh = plsc.VectorSubcoreMesh(core_axis_name="c", subcore_axis_name="s")


@functools.partial(
    pl.kernel,
    out_type=jax.ShapeDtypeStruct((NC, N_AGG, W), jnp.float32),
    mesh=_sc_mesh,
    scratch_types=[
        pltpu.VMEM((NB, BATCH), jnp.int32),
        pltpu.VMEM((NB, BATCH), jnp.int32),
        [pltpu.VMEM((BATCH, W), jnp.float32) for _ in range(NBUF)],
        [pltpu.SemaphoreType.DMA for _ in range(NBUF)],
        pltpu.VMEM_SHARED((N_AGG, W), jnp.float32),
    ],
    compiler_params=pltpu.CompilerParams(use_tc_tiling_on_sc=False),
)
def _sc_agg(tables, edges, zeros_h, out, src_v, dst_v, rows, gsem, shared):
    c = lax.axis_index("c")
    s = lax.axis_index("s")
    table = tables.at[c]
    pltpu.sync_copy(edges.at[0, s], src_v)
    pltpu.sync_copy(edges.at[1, s], dst_v)
    for b in range(NBUF):
        pltpu.async_copy(table.at[src_v.at[b]], rows[b], gsem[b])
    pltpu.sync_copy(zeros_h, shared.at[pl.ds(s * ZROWS, ZROWS)])
    plsc.subcore_barrier()

    @pl.loop(0, NB, step=NBUF)
    def _steps(j):
        for b in range(NBUF):
            jj = j + b
            pltpu.make_async_copy(table.at[src_v.at[jj]], rows[b],
                                  gsem[b]).wait()
            pltpu.sync_copy(rows[b], shared.at[dst_v.at[jj]], add=True)

            @pl.when(jj + NBUF < NB)
            def _prefetch():
                pltpu.async_copy(table.at[src_v.at[jj + NBUF]], rows[b],
                                 gsem[b])

    plsc.subcore_barrier()
    pltpu.sync_copy(shared.at[pl.ds(s * ZROWS, ZROWS)],
                    out.at[c, pl.ds(s * ZROWS, ZROWS)])


def _sage_bn_relu(parts, h, Wl, bl, Wr, g, b):
    cnt = jnp.maximum(parts[0, :N, HH:HH + 1], 1.0)
    m0 = parts[0, :N, :HH] / cnt
    m1 = parts[1, :N, :HH] / cnt
    z = (jnp.dot(m0, Wl[:HH, :], preferred_element_type=jnp.float32)
         + jnp.dot(m1, Wl[HH:, :], preferred_element_type=jnp.float32)
         + bl[None, :]
         + jnp.dot(h[0, :, :HH], Wr[:HH, :],
                   preferred_element_type=jnp.float32)
         + jnp.dot(h[1, :, :HH], Wr[HH:, :],
                   preferred_element_type=jnp.float32))
    mu = jnp.mean(z, axis=0, keepdims=True)
    var = jnp.mean((z - mu) * (z - mu), axis=0, keepdims=True)
    y = (z - mu) * lax.rsqrt(var + EPS) * g[None, :] + b[None, :]
    return jnp.maximum(y, 0.0)


def _tc_layer_body(parts, h, Wl, bl, Wr, g, b, out):
    y = _sage_bn_relu(parts[...], h[...], Wl[...], bl[...], Wr[...],
                      g[...], b[...])
    lane = lax.broadcasted_iota(jnp.int32, (N, W - HH), 1)
    ones_pad = jnp.where(lane == 0, 1.0, 0.0)
    out[0, :, :HH] = y[:, :HH]
    out[0, :, HH:] = ones_pad
    out[1, :, :HH] = y[:, HH:]
    out[1, :, HH:] = ones_pad


def _tc_last_body(parts, h, Wl, bl, Wr, g, b, wlin, blin, out):
    y = _sage_bn_relu(parts[...], h[...], Wl[...], bl[...], Wr[...],
                      g[...], b[...])
    out[...] = (jnp.dot(y, wlin[...], preferred_element_type=jnp.float32)
                + blin[0])


_tc_layer = pl.pallas_call(
    _tc_layer_body,
    out_shape=jax.ShapeDtypeStruct((NC, N, W), jnp.float32),
)

_tc_last = pl.pallas_call(
    _tc_last_body,
    out_shape=jax.ShapeDtypeStruct((N, 1), jnp.float32),
)


def kernel(x, edge_index, Wl0, bl0, Wr0, gamma0, beta0, Wl1, bl1, Wr1,
           gamma1, beta1, Wl2, bl2, Wr2, gamma2, beta2, Wlin, blin):
    pad = E_PAD - E
    src = jnp.concatenate([edge_index[0], jnp.zeros((pad,), jnp.int32)])
    dst = jnp.concatenate([edge_index[1], jnp.full((pad,), N, jnp.int32)])
    edges = jnp.stack([src, dst]).reshape(2, NS, NB, BATCH)
    zeros_h = jnp.zeros((ZROWS, W), jnp.float32)
    ones_pad = jnp.concatenate(
        [jnp.ones((N, 1), jnp.float32),
         jnp.zeros((N, W - HH - 1), jnp.float32)], axis=1)
    h = jnp.stack([jnp.concatenate([x[:, :HH], ones_pad], axis=1),
                   jnp.concatenate([x[:, HH:], ones_pad], axis=1)])

    parts = _sc_agg(h, edges, zeros_h)
    h = _tc_layer(parts, h, Wl0, bl0, Wr0, gamma0, beta0)
    parts = _sc_agg(h, edges, zeros_h)
    h = _tc_layer(parts, h, Wl1, bl1, Wr1, gamma1, beta1)
    parts = _sc_agg(h, edges, zeros_h)
    out = _tc_last(parts, h, Wl2, bl2, Wr2, gamma2, beta2, Wlin, blin)
    return out[:, 0]
